# Initial kernel scaffold; baseline (speedup 1.0000x reference)
import jax
import jax.numpy as jnp
from jax.experimental import pallas as pl


def kernel(input_, weight):
    raise NotImplementedError("write your pallas kernel here")


# speedup vs baseline: 1.8754x; 1.8754x over previous
import functools

import jax
import jax.numpy as jnp
from jax import lax
from jax.experimental import pallas as pl
from jax.experimental.pallas import tpu as pltpu
from jax.experimental.pallas import tpu_sc as plsc

NC = 2
NS = 16
NW = NC * NS
G = 128
GPB = 4
NBUF = 2


@functools.lru_cache(maxsize=None)
def _build(B, V, D):
    n = B // NW
    ng = n // G
    rows_per_buf = G * GPB
    n_steps = ng // GPB
    assert n_steps >= NBUF and (n_steps - NBUF) % NBUF == 0

    mesh = plsc.VectorSubcoreMesh(
        core_axis_name="c", subcore_axis_name="s",
        num_cores=NC, num_subcores=NS)

    @functools.partial(
        pl.kernel,
        mesh=mesh,
        compiler_params=pltpu.CompilerParams(use_tc_tiling_on_sc=False),
        out_type=jax.ShapeDtypeStruct((B, D), jnp.float32),
        scratch_types=[
            pltpu.VMEM((ng, G), jnp.int32),
            pltpu.VMEM((NBUF, rows_per_buf, D), jnp.float32),
            pltpu.SemaphoreType.DMA,
            pltpu.SemaphoreType.DMA,
            pltpu.SemaphoreType.DMA,
            pltpu.SemaphoreType.DMA,
        ],
    )
    def emb(idx_hbm, table_hbm, out_hbm, idx_v, rows_v, g0, g1, o0, o1):
        gsem = (g0, g1)
        osem = (o0, o1)
        wid = lax.axis_index("s") * NC + lax.axis_index("c")
        row_base = wid * n

        pltpu.sync_copy(idx_hbm.at[wid], idx_v)

        def fire_gathers(step, b):
            for j in range(GPB):
                pltpu.async_copy(
                    table_hbm.at[idx_v.at[step * GPB + j]],
                    rows_v.at[b, pl.ds(j * G, G)],
                    gsem[b])

        def wait_gathers(b):
            pltpu.make_async_copy(
                out_hbm.at[pl.ds(0, rows_per_buf)], rows_v.at[b],
                gsem[b]).wait()

        def fire_out(step, b):
            pltpu.async_copy(
                rows_v.at[b],
                out_hbm.at[pl.ds(row_base + step * rows_per_buf, rows_per_buf)],
                osem[b])

        def wait_out(b):
            pltpu.make_async_copy(
                out_hbm.at[pl.ds(0, rows_per_buf)], rows_v.at[b],
                osem[b]).wait()

        for b in range(NBUF):
            fire_gathers(b, b)

        @pl.loop(0, (n_steps - NBUF) // NBUF)
        def _(p):
            for b in range(NBUF):
                step = p * NBUF + b
                wait_gathers(b)
                fire_out(step, b)
                wait_out(b)
                fire_gathers(step + NBUF, b)

        for b in range(NBUF):
            step = n_steps - NBUF + b
            wait_gathers(b)
            fire_out(step, b)
            wait_out(b)

    return emb


def kernel(input_, weight):
    V, D = weight.shape
    orig_shape = input_.shape
    idx = input_.reshape(-1).astype(jnp.int32)
    B = idx.shape[0]
    chunk = NW * G * GPB * NBUF
    pad = (-B) % chunk
    if pad:
        idx = jnp.concatenate([idx, jnp.zeros((pad,), jnp.int32)])
    Bp = B + pad
    idx3 = idx.reshape(NW, Bp // (NW * G), G)
    out = _build(Bp, V, D)(idx3, weight)
    if pad:
        out = out[:B]
    return out.reshape(orig_shape + (D,))


# speedup vs baseline: 1.8856x; 1.0054x over previous
import functools

import jax
import jax.numpy as jnp
from jax import lax
from jax.experimental import pallas as pl
from jax.experimental.pallas import tpu as pltpu
from jax.experimental.pallas import tpu_sc as plsc

NC = 2
NS = 16
NW = NC * NS
G = 128
GPB = 4
NBUF = 3


@functools.lru_cache(maxsize=None)
def _build(B, V, D):
    n = B // NW
    ng = n // G
    rows_per_buf = G * GPB
    n_steps = ng // GPB
    assert n_steps >= NBUF + 1

    mesh = plsc.VectorSubcoreMesh(
        core_axis_name="c", subcore_axis_name="s",
        num_cores=NC, num_subcores=NS)

    @functools.partial(
        pl.kernel,
        mesh=mesh,
        compiler_params=pltpu.CompilerParams(use_tc_tiling_on_sc=False),
        out_type=jax.ShapeDtypeStruct((B, D), jnp.float32),
        scratch_types=[
            pltpu.VMEM((ng, G), jnp.int32),
            pltpu.VMEM((NBUF, rows_per_buf, D), jnp.float32),
            pltpu.SemaphoreType.DMA,
            pltpu.SemaphoreType.DMA,
            pltpu.SemaphoreType.DMA,
            pltpu.SemaphoreType.DMA,
            pltpu.SemaphoreType.DMA,
            pltpu.SemaphoreType.DMA,
        ],
    )
    def emb(idx_hbm, table_hbm, out_hbm, idx_v, rows_v, g0, g1, g2, o0, o1, o2):
        gsem = (g0, g1, g2)
        osem = (o0, o1, o2)
        wid = lax.axis_index("s") * NC + lax.axis_index("c")
        row_base = wid * n

        pltpu.sync_copy(idx_hbm.at[wid], idx_v)

        def fire_gathers(step, b):
            for j in range(GPB):
                pltpu.async_copy(
                    table_hbm.at[idx_v.at[step * GPB + j]],
                    rows_v.at[b, pl.ds(j * G, G)],
                    gsem[b])

        def wait_gathers(b):
            pltpu.make_async_copy(
                out_hbm.at[pl.ds(0, rows_per_buf)], rows_v.at[b],
                gsem[b]).wait()

        def fire_out(step, b):
            pltpu.async_copy(
                rows_v.at[b],
                out_hbm.at[pl.ds(row_base + step * rows_per_buf, rows_per_buf)],
                osem[b])

        def wait_out(b):
            pltpu.make_async_copy(
                out_hbm.at[pl.ds(0, rows_per_buf)], rows_v.at[b],
                osem[b]).wait()

        def step_body(s, b, fire, wait_prev_out=True):
            wait_gathers(b)
            fire_out(s, b)
            if wait_prev_out:
                wait_out((b - 1) % NBUF)
            if fire:
                fire_gathers(s + NBUF - 1, (b - 1) % NBUF)

        for t in range(NBUF - 1):
            fire_gathers(t, t)

        step_body(0, 0, True, wait_prev_out=False)

        last_fire = n_steps - NBUF
        K = (last_fire - 1 + 1) // NBUF if last_fire >= 1 else 0
        if K > 0:
            @pl.loop(0, K)
            def _(p):
                for boff in range(NBUF):
                    s = 1 + p * NBUF + boff
                    step_body(s, (1 + boff) % NBUF, True)

        for s in range(1 + K * NBUF, last_fire + 1):
            step_body(s, s % NBUF, True)

        for s in range(max(last_fire + 1, 1), n_steps):
            step_body(s, s % NBUF, False)

        wait_out((n_steps - 1) % NBUF)

    return emb


def kernel(input_, weight):
    V, D = weight.shape
    orig_shape = input_.shape
    idx = input_.reshape(-1).astype(jnp.int32)
    B = idx.shape[0]
    chunk = NW * G * GPB
    pad = (-B) % chunk
    if pad:
        idx = jnp.concatenate([idx, jnp.zeros((pad,), jnp.int32)])
    Bp = B + pad
    idx3 = idx.reshape(NW, Bp // (NW * G), G)
    out = _build(Bp, V, D)(idx3, weight)
    if pad:
        out = out[:B]
    return out.reshape(orig_shape + (D,))
